# baseline (speedup 1.0000x reference)
import functools

import jax
import jax.numpy as jnp
from jax import lax
from jax.experimental import pallas as pl
from jax.experimental.pallas import tpu as pltpu
from jax.experimental.pallas import tpu_sc as plsc

B, N, C, H, W = 2, 4, 16, 512, 512
W_SPARSE, W_DENSE, W_SEM = 1.0, 0.05, 1.0

BN = B * N
HB = 128
LOG2E = 1.4426950408889634

NC, NS, L = 2, 16, 16
NW = NC * NS
TOT = BN * H * W
PER_W = TOT // NW
CH = 8192


UNROLL = 8
NCHUNK = PER_W // CH


def _sc_depth_kernel(dp_hbm, sg_hbm, dg_hbm, gt_hbm, out_hbm,
                     dp_v, sg_v, dg_v, gt_v, acc_v, sem0, sem1):
    wid = lax.axis_index("s") * NC + lax.axis_index("c")
    base = wid * PER_W
    sems = (sem0, sem1)

    def issue(k, slot):
        off = base + k * CH
        pltpu.async_copy(dp_hbm.at[pl.ds(off, CH)], dp_v.at[slot], sems[slot])
        pltpu.async_copy(sg_hbm.at[pl.ds(off, CH)], sg_v.at[slot], sems[slot])
        pltpu.async_copy(dg_hbm.at[pl.ds(off, CH)], dg_v.at[slot], sems[slot])
        pltpu.async_copy(gt_hbm.at[pl.ds(off, CH)], gt_v.at[slot], sems[slot])

    def drain(slot):
        pltpu.make_async_copy(dp_hbm.at[pl.ds(0, CH)], dp_v.at[slot], sems[slot]).wait()
        pltpu.make_async_copy(sg_hbm.at[pl.ds(0, CH)], sg_v.at[slot], sems[slot]).wait()
        pltpu.make_async_copy(dg_hbm.at[pl.ds(0, CH)], dg_v.at[slot], sems[slot]).wait()
        pltpu.make_async_copy(gt_hbm.at[pl.ds(0, CH)], gt_v.at[slot], sems[slot]).wait()

    def compute(slot, accs):
        def vec_body(j, accs):
            l1s, cs, l1d, cd = accs
            jb = j * (L * UNROLL)
            for u in range(UNROLL):
                o = jb + u * L
                d = dp_v[slot, pl.ds(o, L)]
                s = sg_v[slot, pl.ds(o, L)]
                de = dg_v[slot, pl.ds(o, L)]
                g = gt_v[slot, pl.ds(o, L)]
                m0 = g > 0
                mss = jnp.logical_and(m0, s > 0.0)
                msd = jnp.logical_and(m0, de > 0.0)
                l1s = l1s + jnp.where(mss, jnp.abs(d - s), 0.0)
                cs = cs + jnp.where(mss, 1.0, 0.0)
                l1d = l1d + jnp.where(msd, jnp.abs(d - de), 0.0)
                cd = cd + jnp.where(msd, 1.0, 0.0)
            return l1s, cs, l1d, cd

        return lax.fori_loop(0, CH // (L * UNROLL), vec_body, accs)

    issue(0, 0)

    def pair_body(i, accs):
        k0 = 2 * i
        issue(k0 + 1, 1)
        drain(0)
        accs = compute(0, accs)

        @pl.when(k0 + 2 < NCHUNK)
        def _prefetch():
            issue(k0 + 2, 0)

        drain(1)
        return compute(1, accs)

    z = jnp.zeros((L,), jnp.float32)
    l1s, cs, l1d, cd = lax.fori_loop(0, NCHUNK // 2, pair_body, (z, z, z, z))
    acc_v[0] = l1s
    acc_v[1] = cs
    acc_v[2] = l1d
    acc_v[3] = cd
    pltpu.sync_copy(acc_v, out_hbm.at[wid])


_sc_depth = functools.partial(
    pl.kernel,
    out_type=jax.ShapeDtypeStruct((NW, 4, L), jnp.float32),
    mesh=plsc.VectorSubcoreMesh(
        core_axis_name="c", subcore_axis_name="s", num_cores=NC,
        num_subcores=NS),
    scratch_types=[
        pltpu.VMEM((2, CH), jnp.float32),
        pltpu.VMEM((2, CH), jnp.float32),
        pltpu.VMEM((2, CH), jnp.float32),
        pltpu.VMEM((2, CH), jnp.int32),
        pltpu.VMEM((4, L), jnp.float32),
        pltpu.SemaphoreType.DMA,
        pltpu.SemaphoreType.DMA,
    ],
)(_sc_depth_kernel)


def _ce_kernel(sp_ref, gt_ref, out_ref):
    gt = gt_ref[0]

    m = sp_ref[0, 0]
    for c in range(1, C):
        m = jnp.maximum(m, sp_ref[0, c])

    mscaled = m * LOG2E
    e = jnp.zeros((HB, W), jnp.float32)
    for c in range(C):
        e = e + jnp.exp2(sp_ref[0, c] * LOG2E - mscaled)

    b0 = (gt & 1) != 0
    b1 = (gt & 2) != 0
    b2 = (gt & 4) != 0
    b3 = (gt & 8) != 0
    lvl = [jnp.where(b0, sp_ref[0, 2 * i + 1], sp_ref[0, 2 * i]) for i in range(8)]
    lvl = [jnp.where(b1, lvl[2 * i + 1], lvl[2 * i]) for i in range(4)]
    lvl = [jnp.where(b2, lvl[2 * i + 1], lvl[2 * i]) for i in range(2)]
    logit_gt = jnp.where(b3, lvl[1], lvl[0])
    nll = jnp.log(e) + m - logit_gt

    mf = (gt > 0).astype(jnp.float32)
    nll_sum = jnp.sum(nll * mf)
    cnt = jnp.sum(mf)

    lane = lax.broadcasted_iota(jnp.int32, (1, 128), 1)
    vec = (jnp.where(lane == 0, nll_sum, 0.0)
           + jnp.where(lane == 1, cnt, 0.0))

    first = jnp.logical_and(pl.program_id(0) == 0, pl.program_id(1) == 0)

    @pl.when(first)
    def _init():
        out_ref[...] = vec

    @pl.when(jnp.logical_not(first))
    def _acc():
        out_ref[...] += vec


def kernel(depth_pred, semantic_pred, sparse_depth_gt, dense_depth_gt, semantic_gt):
    sp = semantic_pred.reshape(BN, C, H, W)
    gt = semantic_gt.reshape(BN, H, W).astype(jnp.int32)
    dp = depth_pred.reshape(TOT)
    sg = sparse_depth_gt.reshape(TOT)
    dg = dense_depth_gt.reshape(TOT)
    gt_flat = gt.reshape(TOT)

    sc_part = _sc_depth(dp, sg, dg, gt_flat)

    nh = H // HB
    acc = pl.pallas_call(
        _ce_kernel,
        grid=(BN, nh),
        in_specs=[
            pl.BlockSpec((1, C, HB, W), lambda b, h: (b, 0, h, 0)),
            pl.BlockSpec((1, HB, W), lambda b, h: (b, h, 0)),
        ],
        out_specs=pl.BlockSpec((1, 128), lambda b, h: (0, 0)),
        out_shape=jax.ShapeDtypeStruct((1, 128), jnp.float32),
    )(sp, gt)

    depth_sums = jnp.sum(sc_part, axis=(0, 2))
    l1s, cnt_s, l1d, cnt_d = (depth_sums[0], depth_sums[1],
                              depth_sums[2], depth_sums[3])
    nll_sum, cnt = acc[0, 0], acc[0, 1]

    l_d = jnp.where(cnt_s > 0, l1s / jnp.maximum(cnt_s, 1.0), 0.0)
    l_pd = jnp.where(cnt_d > 0, l1d / jnp.maximum(cnt_d, 1.0), 0.0)
    l_sem = jnp.where(cnt > 0, nll_sum / jnp.maximum(cnt, 1.0), 0.0)
    return W_SPARSE * l_d + W_DENSE * l_pd + W_SEM * l_sem


# speedup vs baseline: 1.3938x; 1.3938x over previous
import functools

import jax
import jax.numpy as jnp
from jax import lax
from jax.experimental import pallas as pl
from jax.experimental.pallas import tpu as pltpu
from jax.experimental.pallas import tpu_sc as plsc

B, N, C, H, W = 2, 4, 16, 512, 512
W_SPARSE, W_DENSE, W_SEM = 1.0, 0.05, 1.0

BN = B * N
HB = 128
LOG2E = 1.4426950408889634

NC, NS, L = 2, 16, 16
NW = NC * NS
TOT = BN * H * W
PER_W = TOT // NW
CH = 8192


ROWS = BN * H
RPW = ROWS // NW
CHR = 16
NCHUNK = RPW // CHR


def _sc_depth_kernel(dp_hbm, sg_hbm, dg_hbm, gt_hbm, out_hbm,
                     dp_v, sg_v, dg_v, gt_v, acc_v, sem0, sem1):
    wid = lax.axis_index("s") * NC + lax.axis_index("c")
    base = wid * RPW
    sems = (sem0, sem1)

    def issue(k, slot):
        r0 = base + k * CHR
        pltpu.async_copy(dp_hbm.at[pl.ds(r0, CHR)], dp_v.at[slot], sems[slot])
        pltpu.async_copy(sg_hbm.at[pl.ds(r0, CHR)], sg_v.at[slot], sems[slot])
        pltpu.async_copy(dg_hbm.at[pl.ds(r0, CHR)], dg_v.at[slot], sems[slot])
        pltpu.async_copy(gt_hbm.at[pl.ds(r0, CHR)], gt_v.at[slot], sems[slot])

    def drain(slot):
        pltpu.make_async_copy(dp_hbm.at[pl.ds(0, CHR)], dp_v.at[slot], sems[slot]).wait()
        pltpu.make_async_copy(sg_hbm.at[pl.ds(0, CHR)], sg_v.at[slot], sems[slot]).wait()
        pltpu.make_async_copy(dg_hbm.at[pl.ds(0, CHR)], dg_v.at[slot], sems[slot]).wait()
        pltpu.make_async_copy(gt_hbm.at[pl.ds(0, CHR)], gt_v.at[slot], sems[slot]).wait()

    def compute(slot, accs):
        def vec_body(j, accs):
            l1s, cs, l1d, cd = accs
            o = j * L
            for r in range(CHR):
                d = dp_v[slot, r, pl.ds(o, L)]
                s = sg_v[slot, r, pl.ds(o, L)]
                de = dg_v[slot, r, pl.ds(o, L)]
                g = gt_v[slot, r, pl.ds(o, L)]
                m0 = g > 0
                mss = jnp.logical_and(m0, s > 0.0)
                msd = jnp.logical_and(m0, de > 0.0)
                l1s = l1s + jnp.where(mss, jnp.abs(d - s), 0.0)
                cs = cs + jnp.where(mss, 1.0, 0.0)
                l1d = l1d + jnp.where(msd, jnp.abs(d - de), 0.0)
                cd = cd + jnp.where(msd, 1.0, 0.0)
            return l1s, cs, l1d, cd

        return lax.fori_loop(0, W // L, vec_body, accs)

    issue(0, 0)

    def pair_body(i, accs):
        k0 = 2 * i
        issue(k0 + 1, 1)
        drain(0)
        accs = compute(0, accs)

        @pl.when(k0 + 2 < NCHUNK)
        def _prefetch():
            issue(k0 + 2, 0)

        drain(1)
        return compute(1, accs)

    z = jnp.zeros((L,), jnp.float32)
    l1s, cs, l1d, cd = lax.fori_loop(0, NCHUNK // 2, pair_body, (z, z, z, z))
    acc_v[0] = l1s
    acc_v[1] = cs
    acc_v[2] = l1d
    acc_v[3] = cd
    pltpu.sync_copy(acc_v, out_hbm.at[wid])


_sc_depth = functools.partial(
    pl.kernel,
    out_type=jax.ShapeDtypeStruct((NW, 4, L), jnp.float32),
    mesh=plsc.VectorSubcoreMesh(
        core_axis_name="c", subcore_axis_name="s", num_cores=NC,
        num_subcores=NS),
    compiler_params=pltpu.CompilerParams(use_tc_tiling_on_sc=True),
    scratch_types=[
        pltpu.VMEM((2, CHR, W), jnp.float32),
        pltpu.VMEM((2, CHR, W), jnp.float32),
        pltpu.VMEM((2, CHR, W), jnp.float32),
        pltpu.VMEM((2, CHR, W), jnp.int32),
        pltpu.VMEM((4, L), jnp.float32),
        pltpu.SemaphoreType.DMA,
        pltpu.SemaphoreType.DMA,
    ],
)(_sc_depth_kernel)


def _ce_kernel(sp_ref, gt_ref, out_ref):
    gt = gt_ref[0]

    m = sp_ref[0, 0]
    for c in range(1, C):
        m = jnp.maximum(m, sp_ref[0, c])

    mscaled = m * LOG2E
    e = jnp.zeros((HB, W), jnp.float32)
    for c in range(C):
        e = e + jnp.exp2(sp_ref[0, c] * LOG2E - mscaled)

    b0 = (gt & 1) != 0
    b1 = (gt & 2) != 0
    b2 = (gt & 4) != 0
    b3 = (gt & 8) != 0
    lvl = [jnp.where(b0, sp_ref[0, 2 * i + 1], sp_ref[0, 2 * i]) for i in range(8)]
    lvl = [jnp.where(b1, lvl[2 * i + 1], lvl[2 * i]) for i in range(4)]
    lvl = [jnp.where(b2, lvl[2 * i + 1], lvl[2 * i]) for i in range(2)]
    logit_gt = jnp.where(b3, lvl[1], lvl[0])
    nll = jnp.log(e) + m - logit_gt

    mf = (gt > 0).astype(jnp.float32)
    nll_sum = jnp.sum(nll * mf)
    cnt = jnp.sum(mf)

    lane = lax.broadcasted_iota(jnp.int32, (1, 128), 1)
    vec = (jnp.where(lane == 0, nll_sum, 0.0)
           + jnp.where(lane == 1, cnt, 0.0))

    first = jnp.logical_and(pl.program_id(0) == 0, pl.program_id(1) == 0)

    @pl.when(first)
    def _init():
        out_ref[...] = vec

    @pl.when(jnp.logical_not(first))
    def _acc():
        out_ref[...] += vec


def kernel(depth_pred, semantic_pred, sparse_depth_gt, dense_depth_gt, semantic_gt):
    sp = semantic_pred.reshape(BN, C, H, W)
    gt = semantic_gt.reshape(BN, H, W).astype(jnp.int32)
    dp = depth_pred.reshape(ROWS, W)
    sg = sparse_depth_gt.reshape(ROWS, W)
    dg = dense_depth_gt.reshape(ROWS, W)
    gt2 = gt.reshape(ROWS, W)

    sc_part = _sc_depth(dp, sg, dg, gt2)

    nh = H // HB
    acc = pl.pallas_call(
        _ce_kernel,
        grid=(BN, nh),
        in_specs=[
            pl.BlockSpec((1, C, HB, W), lambda b, h: (b, 0, h, 0)),
            pl.BlockSpec((1, HB, W), lambda b, h: (b, h, 0)),
        ],
        out_specs=pl.BlockSpec((1, 128), lambda b, h: (0, 0)),
        out_shape=jax.ShapeDtypeStruct((1, 128), jnp.float32),
    )(sp, gt)

    depth_sums = jnp.sum(sc_part, axis=(0, 2))
    l1s, cnt_s, l1d, cnt_d = (depth_sums[0], depth_sums[1],
                              depth_sums[2], depth_sums[3])
    nll_sum, cnt = acc[0, 0], acc[0, 1]

    l_d = jnp.where(cnt_s > 0, l1s / jnp.maximum(cnt_s, 1.0), 0.0)
    l_pd = jnp.where(cnt_d > 0, l1d / jnp.maximum(cnt_d, 1.0), 0.0)
    l_sem = jnp.where(cnt > 0, nll_sum / jnp.maximum(cnt, 1.0), 0.0)
    return W_SPARSE * l_d + W_DENSE * l_pd + W_SEM * l_sem


# speedup vs baseline: 1.4539x; 1.0431x over previous
import functools

import jax
import jax.numpy as jnp
from jax import lax
from jax.experimental import pallas as pl
from jax.experimental.pallas import tpu as pltpu
from jax.experimental.pallas import tpu_sc as plsc

B, N, C, H, W = 2, 4, 16, 512, 512
W_SPARSE, W_DENSE, W_SEM = 1.0, 0.05, 1.0

BN = B * N
HB = 128
LOG2E = 1.4426950408889634

NC, NS, L = 2, 16, 16
NW = NC * NS
TOT = BN * H * W
PER_W = TOT // NW
CH = 8192


ROWS = BN * H
RPW = ROWS // NW
CHR = 16
NCHUNK = RPW // CHR


def _sc_depth_kernel(dp_hbm, sg_hbm, dg_hbm, gt_hbm, out_hbm,
                     dp_v, sg_v, dg_v, gt_v, acc_v, sem0, sem1):
    wid = lax.axis_index("s") * NC + lax.axis_index("c")
    base = wid * RPW
    sems = (sem0, sem1)

    def issue(k, slot):
        r0 = base + k * CHR
        pltpu.async_copy(dp_hbm.at[pl.ds(r0, CHR)], dp_v.at[slot], sems[slot])
        pltpu.async_copy(sg_hbm.at[pl.ds(r0, CHR)], sg_v.at[slot], sems[slot])
        pltpu.async_copy(dg_hbm.at[pl.ds(r0, CHR)], dg_v.at[slot], sems[slot])
        pltpu.async_copy(gt_hbm.at[pl.ds(r0, CHR)], gt_v.at[slot], sems[slot])

    def drain(slot):
        pltpu.make_async_copy(dp_hbm.at[pl.ds(0, CHR)], dp_v.at[slot], sems[slot]).wait()
        pltpu.make_async_copy(sg_hbm.at[pl.ds(0, CHR)], sg_v.at[slot], sems[slot]).wait()
        pltpu.make_async_copy(dg_hbm.at[pl.ds(0, CHR)], dg_v.at[slot], sems[slot]).wait()
        pltpu.make_async_copy(gt_hbm.at[pl.ds(0, CHR)], gt_v.at[slot], sems[slot]).wait()

    def compute(slot, accs):
        def vec_body(j, accs):
            l1s, cs, l1d, cd = accs
            o = j * L
            for r in range(CHR):
                d = dp_v[slot, r, pl.ds(o, L)]
                s = sg_v[slot, r, pl.ds(o, L)]
                de = dg_v[slot, r, pl.ds(o, L)]
                g = gt_v[slot, r, pl.ds(o, L)]
                m0 = g > 0
                mss = jnp.logical_and(m0, s > 0.0)
                msd = jnp.logical_and(m0, de > 0.0)
                l1s = l1s + jnp.where(mss, jnp.abs(d - s), 0.0)
                cs = cs + jnp.where(mss, 1.0, 0.0)
                l1d = l1d + jnp.where(msd, jnp.abs(d - de), 0.0)
                cd = cd + jnp.where(msd, 1.0, 0.0)
            return l1s, cs, l1d, cd

        return lax.fori_loop(0, W // L, vec_body, accs)

    issue(0, 0)

    def pair_body(i, accs):
        k0 = 2 * i
        issue(k0 + 1, 1)
        drain(0)
        accs = compute(0, accs)

        @pl.when(k0 + 2 < NCHUNK)
        def _prefetch():
            issue(k0 + 2, 0)

        drain(1)
        return compute(1, accs)

    z = jnp.zeros((L,), jnp.float32)
    l1s, cs, l1d, cd = lax.fori_loop(0, NCHUNK // 2, pair_body, (z, z, z, z))
    acc_v[0] = l1s
    acc_v[1] = cs
    acc_v[2] = l1d
    acc_v[3] = cd
    pltpu.sync_copy(acc_v, out_hbm.at[wid])


_sc_depth = functools.partial(
    pl.kernel,
    out_type=jax.ShapeDtypeStruct((NW, 4, L), jnp.float32),
    mesh=plsc.VectorSubcoreMesh(
        core_axis_name="c", subcore_axis_name="s", num_cores=NC,
        num_subcores=NS),
    compiler_params=pltpu.CompilerParams(use_tc_tiling_on_sc=True),
    scratch_types=[
        pltpu.VMEM((2, CHR, W), jnp.float32),
        pltpu.VMEM((2, CHR, W), jnp.float32),
        pltpu.VMEM((2, CHR, W), jnp.float32),
        pltpu.VMEM((2, CHR, W), jnp.int32),
        pltpu.VMEM((4, L), jnp.float32),
        pltpu.SemaphoreType.DMA,
        pltpu.SemaphoreType.DMA,
    ],
)(_sc_depth_kernel)


def _ce_kernel(sp_ref, gt_ref, out_ref):
    gt = gt_ref[0]

    x = sp_ref[0]
    m = jnp.max(x, axis=0)
    e = jnp.sum(jnp.exp2(x * LOG2E - (m * LOG2E)[None]), axis=0)

    b0 = (gt & 1) != 0
    b1 = (gt & 2) != 0
    b2 = (gt & 4) != 0
    b3 = (gt & 8) != 0
    lvl = [jnp.where(b0, sp_ref[0, 2 * i + 1], sp_ref[0, 2 * i]) for i in range(8)]
    lvl = [jnp.where(b1, lvl[2 * i + 1], lvl[2 * i]) for i in range(4)]
    lvl = [jnp.where(b2, lvl[2 * i + 1], lvl[2 * i]) for i in range(2)]
    logit_gt = jnp.where(b3, lvl[1], lvl[0])
    nll = jnp.log(e) + m - logit_gt

    mf = (gt > 0).astype(jnp.float32)
    nll_sum = jnp.sum(nll * mf)
    cnt = jnp.sum(mf)

    lane = lax.broadcasted_iota(jnp.int32, (1, 128), 1)
    vec = (jnp.where(lane == 0, nll_sum, 0.0)
           + jnp.where(lane == 1, cnt, 0.0))

    first = jnp.logical_and(pl.program_id(0) == 0, pl.program_id(1) == 0)

    @pl.when(first)
    def _init():
        out_ref[...] = vec

    @pl.when(jnp.logical_not(first))
    def _acc():
        out_ref[...] += vec


def kernel(depth_pred, semantic_pred, sparse_depth_gt, dense_depth_gt, semantic_gt):
    sp = semantic_pred.reshape(BN, C, H, W)
    gt = semantic_gt.reshape(BN, H, W).astype(jnp.int32)
    dp = depth_pred.reshape(ROWS, W)
    sg = sparse_depth_gt.reshape(ROWS, W)
    dg = dense_depth_gt.reshape(ROWS, W)
    gt2 = gt.reshape(ROWS, W)

    sc_part = _sc_depth(dp, sg, dg, gt2)

    nh = H // HB
    acc = pl.pallas_call(
        _ce_kernel,
        grid=(BN, nh),
        in_specs=[
            pl.BlockSpec((1, C, HB, W), lambda b, h: (b, 0, h, 0)),
            pl.BlockSpec((1, HB, W), lambda b, h: (b, h, 0)),
        ],
        out_specs=pl.BlockSpec((1, 128), lambda b, h: (0, 0)),
        out_shape=jax.ShapeDtypeStruct((1, 128), jnp.float32),
    )(sp, gt)

    depth_sums = jnp.sum(sc_part, axis=(0, 2))
    l1s, cnt_s, l1d, cnt_d = (depth_sums[0], depth_sums[1],
                              depth_sums[2], depth_sums[3])
    nll_sum, cnt = acc[0, 0], acc[0, 1]

    l_d = jnp.where(cnt_s > 0, l1s / jnp.maximum(cnt_s, 1.0), 0.0)
    l_pd = jnp.where(cnt_d > 0, l1d / jnp.maximum(cnt_d, 1.0), 0.0)
    l_sem = jnp.where(cnt > 0, nll_sum / jnp.maximum(cnt, 1.0), 0.0)
    return W_SPARSE * l_d + W_DENSE * l_pd + W_SEM * l_sem


# speedup vs baseline: 1.4755x; 1.0149x over previous
import functools

import jax
import jax.numpy as jnp
from jax import lax
from jax.experimental import pallas as pl
from jax.experimental.pallas import tpu as pltpu
from jax.experimental.pallas import tpu_sc as plsc

B, N, C, H, W = 2, 4, 16, 512, 512
W_SPARSE, W_DENSE, W_SEM = 1.0, 0.05, 1.0

BN = B * N
HB = 128
LOG2E = 1.4426950408889634

NC, NS, L = 2, 16, 16
NW = NC * NS
TOT = BN * H * W
PER_W = TOT // NW
CH = 8192


ROWS = BN * H
RPW = ROWS // NW
CHR = 16
NCHUNK = RPW // CHR


def _sc_depth_kernel(dp_hbm, sg_hbm, dg_hbm, gt_hbm, out_hbm,
                     dp_v, sg_v, dg_v, gt_v, acc_v, sem0, sem1):
    wid = lax.axis_index("s") * NC + lax.axis_index("c")
    base = wid * RPW
    sems = (sem0, sem1)

    def issue(k, slot):
        r0 = base + k * CHR
        pltpu.async_copy(dp_hbm.at[pl.ds(r0, CHR)], dp_v.at[slot], sems[slot])
        pltpu.async_copy(sg_hbm.at[pl.ds(r0, CHR)], sg_v.at[slot], sems[slot])
        pltpu.async_copy(dg_hbm.at[pl.ds(r0, CHR)], dg_v.at[slot], sems[slot])
        pltpu.async_copy(gt_hbm.at[pl.ds(r0, CHR)], gt_v.at[slot], sems[slot])

    def drain(slot):
        pltpu.make_async_copy(dp_hbm.at[pl.ds(0, CHR)], dp_v.at[slot], sems[slot]).wait()
        pltpu.make_async_copy(sg_hbm.at[pl.ds(0, CHR)], sg_v.at[slot], sems[slot]).wait()
        pltpu.make_async_copy(dg_hbm.at[pl.ds(0, CHR)], dg_v.at[slot], sems[slot]).wait()
        pltpu.make_async_copy(gt_hbm.at[pl.ds(0, CHR)], gt_v.at[slot], sems[slot]).wait()

    def compute(slot, accs):
        def vec_body(j, accs):
            l1s, cs, l1d, cd = accs
            o = j * L
            for r in range(CHR):
                d = dp_v[slot, r, pl.ds(o, L)]
                s = sg_v[slot, r, pl.ds(o, L)]
                de = dg_v[slot, r, pl.ds(o, L)]
                g = gt_v[slot, r, pl.ds(o, L)]
                m0 = g > 0
                mss = jnp.logical_and(m0, s > 0.0)
                msd = jnp.logical_and(m0, de > 0.0)
                l1s = l1s + jnp.where(mss, jnp.abs(d - s), 0.0)
                cs = cs + jnp.where(mss, 1.0, 0.0)
                l1d = l1d + jnp.where(msd, jnp.abs(d - de), 0.0)
                cd = cd + jnp.where(msd, 1.0, 0.0)
            return l1s, cs, l1d, cd

        return lax.fori_loop(0, W // L, vec_body, accs)

    issue(0, 0)

    def pair_body(i, accs):
        k0 = 2 * i
        issue(k0 + 1, 1)
        drain(0)
        accs = compute(0, accs)

        @pl.when(k0 + 2 < NCHUNK)
        def _prefetch():
            issue(k0 + 2, 0)

        drain(1)
        return compute(1, accs)

    z = jnp.zeros((L,), jnp.float32)
    l1s, cs, l1d, cd = lax.fori_loop(0, NCHUNK // 2, pair_body, (z, z, z, z))
    acc_v[0] = l1s
    acc_v[1] = cs
    acc_v[2] = l1d
    acc_v[3] = cd
    pltpu.sync_copy(acc_v, out_hbm.at[wid])


_sc_depth = functools.partial(
    pl.kernel,
    out_type=jax.ShapeDtypeStruct((NW, 4, L), jnp.float32),
    mesh=plsc.VectorSubcoreMesh(
        core_axis_name="c", subcore_axis_name="s", num_cores=NC,
        num_subcores=NS),
    compiler_params=pltpu.CompilerParams(use_tc_tiling_on_sc=True),
    scratch_types=[
        pltpu.VMEM((2, CHR, W), jnp.float32),
        pltpu.VMEM((2, CHR, W), jnp.float32),
        pltpu.VMEM((2, CHR, W), jnp.float32),
        pltpu.VMEM((2, CHR, W), jnp.int32),
        pltpu.VMEM((4, L), jnp.float32),
        pltpu.SemaphoreType.DMA,
        pltpu.SemaphoreType.DMA,
    ],
)(_sc_depth_kernel)


def _ce_kernel(sp_ref, gt_ref, out_ref):
    gt = gt_ref[0]

    x = sp_ref[0]
    xb = x.astype(jnp.bfloat16)
    mb = jnp.max(xb, axis=0)
    eb = jnp.sum(jnp.exp2((xb - mb[None]) * jnp.bfloat16(LOG2E)), axis=0)
    e = eb.astype(jnp.float32)
    m = mb.astype(jnp.float32)

    b0 = (gt & 1) != 0
    b1 = (gt & 2) != 0
    b2 = (gt & 4) != 0
    b3 = (gt & 8) != 0
    lvl = [jnp.where(b0, sp_ref[0, 2 * i + 1], sp_ref[0, 2 * i]) for i in range(8)]
    lvl = [jnp.where(b1, lvl[2 * i + 1], lvl[2 * i]) for i in range(4)]
    lvl = [jnp.where(b2, lvl[2 * i + 1], lvl[2 * i]) for i in range(2)]
    logit_gt = jnp.where(b3, lvl[1], lvl[0])
    nll = jnp.log(e) + m - logit_gt

    mf = (gt > 0).astype(jnp.float32)
    nll_sum = jnp.sum(nll * mf)
    cnt = jnp.sum(mf)

    lane = lax.broadcasted_iota(jnp.int32, (1, 128), 1)
    vec = (jnp.where(lane == 0, nll_sum, 0.0)
           + jnp.where(lane == 1, cnt, 0.0))

    first = jnp.logical_and(pl.program_id(0) == 0, pl.program_id(1) == 0)

    @pl.when(first)
    def _init():
        out_ref[...] = vec

    @pl.when(jnp.logical_not(first))
    def _acc():
        out_ref[...] += vec


def kernel(depth_pred, semantic_pred, sparse_depth_gt, dense_depth_gt, semantic_gt):
    sp = semantic_pred.reshape(BN, C, H, W)
    gt = semantic_gt.reshape(BN, H, W).astype(jnp.int32)
    dp = depth_pred.reshape(ROWS, W)
    sg = sparse_depth_gt.reshape(ROWS, W)
    dg = dense_depth_gt.reshape(ROWS, W)
    gt2 = gt.reshape(ROWS, W)

    sc_part = _sc_depth(dp, sg, dg, gt2)

    nh = H // HB
    acc = pl.pallas_call(
        _ce_kernel,
        grid=(BN, nh),
        in_specs=[
            pl.BlockSpec((1, C, HB, W), lambda b, h: (b, 0, h, 0)),
            pl.BlockSpec((1, HB, W), lambda b, h: (b, h, 0)),
        ],
        out_specs=pl.BlockSpec((1, 128), lambda b, h: (0, 0)),
        out_shape=jax.ShapeDtypeStruct((1, 128), jnp.float32),
    )(sp, gt)

    depth_sums = jnp.sum(sc_part, axis=(0, 2))
    l1s, cnt_s, l1d, cnt_d = (depth_sums[0], depth_sums[1],
                              depth_sums[2], depth_sums[3])
    nll_sum, cnt = acc[0, 0], acc[0, 1]

    l_d = jnp.where(cnt_s > 0, l1s / jnp.maximum(cnt_s, 1.0), 0.0)
    l_pd = jnp.where(cnt_d > 0, l1d / jnp.maximum(cnt_d, 1.0), 0.0)
    l_sem = jnp.where(cnt > 0, nll_sum / jnp.maximum(cnt, 1.0), 0.0)
    return W_SPARSE * l_d + W_DENSE * l_pd + W_SEM * l_sem


# speedup vs baseline: 1.5967x; 1.0822x over previous
import functools

import jax
import jax.numpy as jnp
from jax import lax
from jax.experimental import pallas as pl
from jax.experimental.pallas import tpu as pltpu
from jax.experimental.pallas import tpu_sc as plsc

B, N, C, H, W = 2, 4, 16, 512, 512
W_SPARSE, W_DENSE, W_SEM = 1.0, 0.05, 1.0

BN = B * N
HB = 128
LOG2E = 1.4426950408889634

NC, NS, L = 2, 16, 16
NW = NC * NS
TOT = BN * H * W
PER_W = TOT // NW
CH = 8192


ROWS = BN * H
RPW = ROWS // NW
CHR = 16
NCHUNK = RPW // CHR


def _sc_depth_kernel(dp_hbm, sg_hbm, dg_hbm, gt_hbm, out_hbm,
                     dp_v, sg_v, dg_v, gt_v, acc_v, sem0, sem1):
    wid = lax.axis_index("s") * NC + lax.axis_index("c")
    base = wid * RPW
    sems = (sem0, sem1)

    def issue(k, slot):
        r0 = base + k * CHR
        pltpu.async_copy(dp_hbm.at[pl.ds(r0, CHR)], dp_v.at[slot], sems[slot])
        pltpu.async_copy(sg_hbm.at[pl.ds(r0, CHR)], sg_v.at[slot], sems[slot])
        pltpu.async_copy(dg_hbm.at[pl.ds(r0, CHR)], dg_v.at[slot], sems[slot])
        pltpu.async_copy(gt_hbm.at[pl.ds(r0, CHR)], gt_v.at[slot], sems[slot])

    def drain(slot):
        pltpu.make_async_copy(dp_hbm.at[pl.ds(0, CHR)], dp_v.at[slot], sems[slot]).wait()
        pltpu.make_async_copy(sg_hbm.at[pl.ds(0, CHR)], sg_v.at[slot], sems[slot]).wait()
        pltpu.make_async_copy(dg_hbm.at[pl.ds(0, CHR)], dg_v.at[slot], sems[slot]).wait()
        pltpu.make_async_copy(gt_hbm.at[pl.ds(0, CHR)], gt_v.at[slot], sems[slot]).wait()

    def compute(slot, accs):
        def vec_body(j, accs):
            l1s, cs, l1d, cd = accs
            o = j * L
            for r in range(CHR):
                d = dp_v[slot, r, pl.ds(o, L)]
                s = sg_v[slot, r, pl.ds(o, L)]
                de = dg_v[slot, r, pl.ds(o, L)]
                g = gt_v[slot, r, pl.ds(o, L)]
                m0 = g > 0
                mss = jnp.logical_and(m0, s > 0.0)
                msd = jnp.logical_and(m0, de > 0.0)
                l1s = l1s + jnp.where(mss, jnp.abs(d - s), 0.0)
                cs = cs + jnp.where(mss, 1.0, 0.0)
                l1d = l1d + jnp.where(msd, jnp.abs(d - de), 0.0)
                cd = cd + jnp.where(msd, 1.0, 0.0)
            return l1s, cs, l1d, cd

        return lax.fori_loop(0, W // L, vec_body, accs)

    issue(0, 0)

    def pair_body(i, accs):
        k0 = 2 * i
        issue(k0 + 1, 1)
        drain(0)
        accs = compute(0, accs)

        @pl.when(k0 + 2 < NCHUNK)
        def _prefetch():
            issue(k0 + 2, 0)

        drain(1)
        return compute(1, accs)

    z = jnp.zeros((L,), jnp.float32)
    l1s, cs, l1d, cd = lax.fori_loop(0, NCHUNK // 2, pair_body, (z, z, z, z))
    acc_v[0] = l1s
    acc_v[1] = cs
    acc_v[2] = l1d
    acc_v[3] = cd
    pltpu.sync_copy(acc_v, out_hbm.at[wid])


_sc_depth = functools.partial(
    pl.kernel,
    out_type=jax.ShapeDtypeStruct((NW, 4, L), jnp.float32),
    mesh=plsc.VectorSubcoreMesh(
        core_axis_name="c", subcore_axis_name="s", num_cores=NC,
        num_subcores=NS),
    compiler_params=pltpu.CompilerParams(use_tc_tiling_on_sc=True),
    scratch_types=[
        pltpu.VMEM((2, CHR, W), jnp.float32),
        pltpu.VMEM((2, CHR, W), jnp.float32),
        pltpu.VMEM((2, CHR, W), jnp.float32),
        pltpu.VMEM((2, CHR, W), jnp.int32),
        pltpu.VMEM((4, L), jnp.float32),
        pltpu.SemaphoreType.DMA,
        pltpu.SemaphoreType.DMA,
    ],
)(_sc_depth_kernel)


def _ce_kernel(sp_ref, gt_ref, out_ref):
    gt = gt_ref[0]

    x = sp_ref[0]
    xb = x.astype(jnp.bfloat16)
    mb = jnp.max(xb, axis=0)
    t = jnp.exp2((xb - mb[None]) * jnp.bfloat16(LOG2E))
    eb = t[0]
    for c in range(1, C):
        eb = eb + t[c]
    e = eb.astype(jnp.float32)
    m = mb.astype(jnp.float32)

    b0 = (gt & 1) != 0
    b1 = (gt & 2) != 0
    b2 = (gt & 4) != 0
    b3 = (gt & 8) != 0
    lvl = [jnp.where(b0, xb[2 * i + 1], xb[2 * i]) for i in range(8)]
    lvl = [jnp.where(b1, lvl[2 * i + 1], lvl[2 * i]) for i in range(4)]
    lvl = [jnp.where(b2, lvl[2 * i + 1], lvl[2 * i]) for i in range(2)]
    logit_gt = jnp.where(b3, lvl[1], lvl[0]).astype(jnp.float32)
    nll = jnp.log(e) + m - logit_gt

    mf = (gt > 0).astype(jnp.float32)
    nll_sum = jnp.sum(nll * mf)
    cnt = jnp.sum(mf)

    lane = lax.broadcasted_iota(jnp.int32, (1, 128), 1)
    vec = (jnp.where(lane == 0, nll_sum, 0.0)
           + jnp.where(lane == 1, cnt, 0.0))

    first = jnp.logical_and(pl.program_id(0) == 0, pl.program_id(1) == 0)

    @pl.when(first)
    def _init():
        out_ref[...] = vec

    @pl.when(jnp.logical_not(first))
    def _acc():
        out_ref[...] += vec


def kernel(depth_pred, semantic_pred, sparse_depth_gt, dense_depth_gt, semantic_gt):
    sp = semantic_pred.reshape(BN, C, H, W)
    gt = semantic_gt.reshape(BN, H, W).astype(jnp.int32)
    dp = depth_pred.reshape(ROWS, W)
    sg = sparse_depth_gt.reshape(ROWS, W)
    dg = dense_depth_gt.reshape(ROWS, W)
    gt2 = gt.reshape(ROWS, W)

    sc_part = _sc_depth(dp, sg, dg, gt2)

    nh = H // HB
    acc = pl.pallas_call(
        _ce_kernel,
        grid=(BN, nh),
        in_specs=[
            pl.BlockSpec((1, C, HB, W), lambda b, h: (b, 0, h, 0)),
            pl.BlockSpec((1, HB, W), lambda b, h: (b, h, 0)),
        ],
        out_specs=pl.BlockSpec((1, 128), lambda b, h: (0, 0)),
        out_shape=jax.ShapeDtypeStruct((1, 128), jnp.float32),
    )(sp, gt)

    depth_sums = jnp.sum(sc_part, axis=(0, 2))
    l1s, cnt_s, l1d, cnt_d = (depth_sums[0], depth_sums[1],
                              depth_sums[2], depth_sums[3])
    nll_sum, cnt = acc[0, 0], acc[0, 1]

    l_d = jnp.where(cnt_s > 0, l1s / jnp.maximum(cnt_s, 1.0), 0.0)
    l_pd = jnp.where(cnt_d > 0, l1d / jnp.maximum(cnt_d, 1.0), 0.0)
    l_sem = jnp.where(cnt > 0, nll_sum / jnp.maximum(cnt, 1.0), 0.0)
    return W_SPARSE * l_d + W_DENSE * l_pd + W_SEM * l_sem


# speedup vs baseline: 1.6375x; 1.0255x over previous
import functools

import jax
import jax.numpy as jnp
from jax import lax
from jax.experimental import pallas as pl
from jax.experimental.pallas import tpu as pltpu
from jax.experimental.pallas import tpu_sc as plsc

B, N, C, H, W = 2, 4, 16, 512, 512
W_SPARSE, W_DENSE, W_SEM = 1.0, 0.05, 1.0

BN = B * N
HB = 128
LOG2E = 1.4426950408889634

NC, NS, L = 2, 16, 16
NW = NC * NS
TOT = BN * H * W
PER_W = TOT // NW
CH = 8192


ROWS = BN * H
RPW = ROWS // NW
CHR = 16
NCHUNK = RPW // CHR


def _sc_depth_kernel(dp_hbm, sg_hbm, dg_hbm, gt_hbm, out_hbm,
                     dp_v, sg_v, dg_v, gt_v, acc_v, sem0, sem1):
    wid = lax.axis_index("s") * NC + lax.axis_index("c")
    base = wid * RPW
    sems = (sem0, sem1)

    def issue(k, slot):
        r0 = base + k * CHR
        pltpu.async_copy(dp_hbm.at[pl.ds(r0, CHR)], dp_v.at[slot], sems[slot])
        pltpu.async_copy(sg_hbm.at[pl.ds(r0, CHR)], sg_v.at[slot], sems[slot])
        pltpu.async_copy(dg_hbm.at[pl.ds(r0, CHR)], dg_v.at[slot], sems[slot])
        pltpu.async_copy(gt_hbm.at[pl.ds(r0, CHR)], gt_v.at[slot], sems[slot])

    def drain(slot):
        pltpu.make_async_copy(dp_hbm.at[pl.ds(0, CHR)], dp_v.at[slot], sems[slot]).wait()
        pltpu.make_async_copy(sg_hbm.at[pl.ds(0, CHR)], sg_v.at[slot], sems[slot]).wait()
        pltpu.make_async_copy(dg_hbm.at[pl.ds(0, CHR)], dg_v.at[slot], sems[slot]).wait()
        pltpu.make_async_copy(gt_hbm.at[pl.ds(0, CHR)], gt_v.at[slot], sems[slot]).wait()

    def compute(slot, accs):
        for rg in range(0, CHR, 4):
            def vec_body(j, accs, rg=rg):
                l1s, cs, l1d, cd = accs
                o = j * L
                for r in range(rg, rg + 4):
                    d = dp_v[slot, r, pl.ds(o, L)]
                    s = sg_v[slot, r, pl.ds(o, L)]
                    de = dg_v[slot, r, pl.ds(o, L)]
                    g = gt_v[slot, r, pl.ds(o, L)]
                    m0 = g > 0
                    mss = jnp.logical_and(m0, s > 0.0)
                    msd = jnp.logical_and(m0, de > 0.0)
                    l1s = l1s + jnp.where(mss, jnp.abs(d - s), 0.0)
                    cs = cs + jnp.where(mss, 1.0, 0.0)
                    l1d = l1d + jnp.where(msd, jnp.abs(d - de), 0.0)
                    cd = cd + jnp.where(msd, 1.0, 0.0)
                return l1s, cs, l1d, cd

            accs = lax.fori_loop(0, W // L, vec_body, accs)
        return accs

    issue(0, 0)

    def pair_body(i, accs):
        k0 = 2 * i
        issue(k0 + 1, 1)
        drain(0)
        accs = compute(0, accs)

        @pl.when(k0 + 2 < NCHUNK)
        def _prefetch():
            issue(k0 + 2, 0)

        drain(1)
        return compute(1, accs)

    z = jnp.zeros((L,), jnp.float32)
    l1s, cs, l1d, cd = lax.fori_loop(0, NCHUNK // 2, pair_body, (z, z, z, z))
    acc_v[0] = l1s
    acc_v[1] = cs
    acc_v[2] = l1d
    acc_v[3] = cd
    pltpu.sync_copy(acc_v, out_hbm.at[wid])


_sc_depth = functools.partial(
    pl.kernel,
    out_type=jax.ShapeDtypeStruct((NW, 4, L), jnp.float32),
    mesh=plsc.VectorSubcoreMesh(
        core_axis_name="c", subcore_axis_name="s", num_cores=NC,
        num_subcores=NS),
    compiler_params=pltpu.CompilerParams(use_tc_tiling_on_sc=True),
    scratch_types=[
        pltpu.VMEM((2, CHR, W), jnp.float32),
        pltpu.VMEM((2, CHR, W), jnp.float32),
        pltpu.VMEM((2, CHR, W), jnp.float32),
        pltpu.VMEM((2, CHR, W), jnp.int32),
        pltpu.VMEM((4, L), jnp.float32),
        pltpu.SemaphoreType.DMA,
        pltpu.SemaphoreType.DMA,
    ],
)(_sc_depth_kernel)


def _ce_kernel(sp_ref, gt_ref, out_ref):
    gt = gt_ref[0]

    x = sp_ref[0]
    xb = x.astype(jnp.bfloat16)
    mb = jnp.max(xb, axis=0)
    t = jnp.exp2((xb - mb[None]) * jnp.bfloat16(LOG2E))
    eb = t[0]
    for c in range(1, C):
        eb = eb + t[c]
    e = eb.astype(jnp.float32)
    m = mb.astype(jnp.float32)

    b0 = (gt & 1) != 0
    b1 = (gt & 2) != 0
    b2 = (gt & 4) != 0
    b3 = (gt & 8) != 0
    lvl = [jnp.where(b0, xb[2 * i + 1], xb[2 * i]) for i in range(8)]
    lvl = [jnp.where(b1, lvl[2 * i + 1], lvl[2 * i]) for i in range(4)]
    lvl = [jnp.where(b2, lvl[2 * i + 1], lvl[2 * i]) for i in range(2)]
    logit_gt = jnp.where(b3, lvl[1], lvl[0]).astype(jnp.float32)
    nll = jnp.log(e) + m - logit_gt

    mf = (gt > 0).astype(jnp.float32)
    nll_sum = jnp.sum(nll * mf)
    cnt = jnp.sum(mf)

    lane = lax.broadcasted_iota(jnp.int32, (1, 128), 1)
    vec = (jnp.where(lane == 0, nll_sum, 0.0)
           + jnp.where(lane == 1, cnt, 0.0))

    first = jnp.logical_and(pl.program_id(0) == 0, pl.program_id(1) == 0)

    @pl.when(first)
    def _init():
        out_ref[...] = vec

    @pl.when(jnp.logical_not(first))
    def _acc():
        out_ref[...] += vec


def kernel(depth_pred, semantic_pred, sparse_depth_gt, dense_depth_gt, semantic_gt):
    sp = semantic_pred.reshape(BN, C, H, W)
    gt = semantic_gt.reshape(BN, H, W).astype(jnp.int32)
    dp = depth_pred.reshape(ROWS, W)
    sg = sparse_depth_gt.reshape(ROWS, W)
    dg = dense_depth_gt.reshape(ROWS, W)
    gt2 = gt.reshape(ROWS, W)

    sc_part = _sc_depth(dp, sg, dg, gt2)

    nh = H // HB
    acc = pl.pallas_call(
        _ce_kernel,
        grid=(BN, nh),
        in_specs=[
            pl.BlockSpec((1, C, HB, W), lambda b, h: (b, 0, h, 0)),
            pl.BlockSpec((1, HB, W), lambda b, h: (b, h, 0)),
        ],
        out_specs=pl.BlockSpec((1, 128), lambda b, h: (0, 0)),
        out_shape=jax.ShapeDtypeStruct((1, 128), jnp.float32),
    )(sp, gt)

    depth_sums = jnp.sum(sc_part, axis=(0, 2))
    l1s, cnt_s, l1d, cnt_d = (depth_sums[0], depth_sums[1],
                              depth_sums[2], depth_sums[3])
    nll_sum, cnt = acc[0, 0], acc[0, 1]

    l_d = jnp.where(cnt_s > 0, l1s / jnp.maximum(cnt_s, 1.0), 0.0)
    l_pd = jnp.where(cnt_d > 0, l1d / jnp.maximum(cnt_d, 1.0), 0.0)
    l_sem = jnp.where(cnt > 0, nll_sum / jnp.maximum(cnt, 1.0), 0.0)
    return W_SPARSE * l_d + W_DENSE * l_pd + W_SEM * l_sem
